# baseline (speedup 1.0000x reference)
import dataclasses
import functools

import jax
import jax.numpy as jnp
from jax import lax
from jax.experimental import pallas as pl
from jax.experimental.pallas import tpu as pltpu
from jax.experimental.pallas import tpu_sc as plsc

K = 9
L = 16
B = 4
N = 20000
G = 64
ROWS = B * G
NC = 2
NS = 16
NW = NC * NS
RPW = ROWS // NW
WPB = NW // B
CHUNKS = N // L
BCH = 16
NBLK = (CHUNKS + BCH - 1) // BCH
PADN = NBLK * BCH * L
NSTR = NBLK * L
GRB = 8
BROW = (NBLK + 1) * L


def _dyn_gather(x, idx):
    dnums = lax.GatherDimensionNumbers(
        offset_dims=(), collapsed_slice_dims=(0,), start_index_map=(0,))
    return lax.gather(x, idx[:, None], dnums, (1,),
                      mode=lax.GatherScatterMode.PROMISE_IN_BOUNDS)


def _sc_topk_body(prep_hbm, gx_hbm, gy_hbm,
                  d2_out, idx_out, cx_out, cy_out, w_out, h_out,
                  cx_v, cy_v, w_v, h_v, gx_v, gy_v,
                  sd2, sidx, scx, scy, sw, sh,
                  top_d_ref, top_i_ref, tmax_ref, bmin_ref, sid_ref,
                  gmin_ref, sem_w, sem_h):
    cid = lax.axis_index("c")
    sid = lax.axis_index("s")
    wid = cid * NS + sid
    b = wid // WPB
    base_elem = wid * (RPW * L)

    cp_w = pltpu.async_copy(prep_hbm.at[b, 2], w_v, sem_w)
    cp_h = pltpu.async_copy(prep_hbm.at[b, 3], h_v, sem_h)
    pltpu.sync_copy(prep_hbm.at[b, 0], cx_v)
    pltpu.sync_copy(prep_hbm.at[b, 1], cy_v)
    pltpu.sync_copy(gx_hbm.at[pl.ds(base_elem, RPW * L)], gx_v)
    pltpu.sync_copy(gy_hbm.at[pl.ds(base_elem, RPW * L)], gy_v)

    lanes = lax.iota(jnp.int32, L)
    last_lane = jnp.full((L,), L - 1, jnp.int32)

    def insert_or_merge(d2, idxc):
        fv = plsc.all_reduce_ffs(d2 < tmax_ref[...])
        xd = _dyn_gather(d2, fv)
        xi = _dyn_gather(idxc, fv)
        td = top_d_ref[...]
        ti = top_i_ref[...]
        sh_i = jnp.maximum(lanes - 1, 0)
        sd_ = jnp.where(lanes == 0, jnp.float32(-jnp.inf), _dyn_gather(td, sh_i))
        si_ = _dyn_gather(ti, sh_i)
        keep = td <= xd
        shift = sd_ > xd
        nd = jnp.where(keep, td, jnp.where(shift, sd_, xd))
        ni = jnp.where(keep, ti, jnp.where(shift, si_, xi))
        top_d_ref[...] = nd
        top_i_ref[...] = ni
        tmax_ref[...] = _dyn_gather(nd, last_lane)

        d2m = jnp.where(lanes == fv, jnp.float32(jnp.inf), d2)

        @pl.when(jnp.any(d2m < tmax_ref[...]))
        def _():
            sort_merge(d2m, idxc)

    def sort_merge(d2, idxc):
        td = top_d_ref[...]
        ti = top_i_ref[...]
        ds_, is_ = plsc.sort_key_val(d2, idxc)
        rd = lax.rev(ds_, (0,))
        ri = lax.rev(is_, (0,))
        takes = td <= rd
        md = jnp.where(takes, td, rd)
        mi = jnp.where(takes, ti, ri)
        nd, ni = plsc.sort_key_val(md, mi)
        top_d_ref[...] = nd
        top_i_ref[...] = ni
        tmax_ref[...] = _dyn_gather(nd, last_lane)

    gxs = [gx_v[pl.ds(g * L, L)] for g in range(RPW)]
    gys = [gy_v[pl.ds(g * L, L)] for g in range(RPW)]
    inf16 = jnp.full((L,), jnp.inf, jnp.float32)

    @pl.loop(0, NBLK, init_carry=tuple(inf16 for _ in range(RPW)))
    def _per_block(k, gmins):
        base0 = k * (BCH * L)
        accs = [None] * RPW
        for c in range(BCH):
            cxc = cx_v[pl.ds(base0 + c * L, L)]
            cyc = cy_v[pl.ds(base0 + c * L, L)]
            for g in range(RPW):
                dx = cxc - gxs[g]
                dy = cyc - gys[g]
                d2 = dx * dx + dy * dy
                accs[g] = d2 if c == 0 else jnp.minimum(accs[g], d2)
        for g in range(RPW):
            bmin_ref[pl.ds(g * BROW + k * L, L)] = accs[g]
        return tuple(jnp.minimum(gm, acc) for gm, acc in zip(gmins, accs))

    gmins = _per_block
    for g in range(RPW):
        gmin_ref[pl.ds(g * L, L)] = gmins[g]
        bmin_ref[pl.ds(g * BROW + NBLK * L, L)] = inf16
    cp_w.wait()
    cp_h.wait()

    @pl.loop(0, RPW)
    def _per_gt(g):
        gxv = gx_v[pl.ds(g * L, L)]
        gyv = gy_v[pl.ds(g * L, L)]
        bbase = g * BROW

        gmin = gmin_ref[pl.ds(g * L, L)]
        seed = plsc.bitcast(
            plsc.bitcast(jnp.full((L,), jnp.max(gmin), jnp.float32),
                         jnp.int32) + 1, jnp.float32)
        top_d_ref[...] = jnp.full((L,), jnp.inf, jnp.float32)
        top_i_ref[...] = jnp.zeros((L,), jnp.int32)
        tmax_ref[...] = seed

        @pl.loop(0, (NBLK + 1) // GRB)
        def _scan_bmin(gi):
            b0 = gi * (GRB * L)
            tm = tmax_ref[...]
            hit = jnp.zeros((L,), jnp.bool_)
            for jj in range(GRB):
                bm = bmin_ref[pl.ds(bbase + b0 + jj * L, L)]
                hit = hit | (bm < tm)

            @pl.when(jnp.any(hit))
            def _():
                for jj in range(GRB):
                    bm = bmin_ref[pl.ds(bbase + b0 + jj * L, L)]

                    @pl.when(jnp.any(bm < tmax_ref[...]))
                    def _():
                        insert_or_merge(bm, b0 + jj * L + lanes)

        sid_ref[pl.ds(0, L)] = top_i_ref[...]
        sid_ref[pl.ds(L, L)] = top_i_ref[...]
        top_d_ref[...] = jnp.full((L,), jnp.inf, jnp.float32)
        top_i_ref[...] = jnp.zeros((L,), jnp.int32)
        tmax_ref[...] = jnp.full((L,), jnp.inf, jnp.float32)

        for j in range(L):
            sv = plsc.load_gather(sid_ref, [jnp.full((L,), L + j, jnp.int32)])
            blk = lax.shift_right_logical(sv, 4)
            lane = jnp.bitwise_and(sv, L - 1)
            eidx = blk * (BCH * L) + lane + lanes * L
            cxg = plsc.load_gather(cx_v, [eidx])
            cyg = plsc.load_gather(cy_v, [eidx])
            dx = cxg - gxv
            dy = cyg - gyv
            d2 = dx * dx + dy * dy

            @pl.when(jnp.any(d2 < tmax_ref[...]))
            def _():
                sort_merge(d2, eidx)

        top_d = top_d_ref[...]
        top_i = top_i_ref[...]
        o = g * L
        sd2[pl.ds(o, L)] = top_d
        sidx[pl.ds(o, L)] = top_i
        scx[pl.ds(o, L)] = plsc.load_gather(cx_v, [top_i])
        scy[pl.ds(o, L)] = plsc.load_gather(cy_v, [top_i])
        sw[pl.ds(o, L)] = plsc.load_gather(w_v, [top_i])
        sh[pl.ds(o, L)] = plsc.load_gather(h_v, [top_i])

    pltpu.sync_copy(sd2, d2_out.at[pl.ds(base_elem, RPW * L)])
    pltpu.sync_copy(sidx, idx_out.at[pl.ds(base_elem, RPW * L)])
    pltpu.sync_copy(scx, cx_out.at[pl.ds(base_elem, RPW * L)])
    pltpu.sync_copy(scy, cy_out.at[pl.ds(base_elem, RPW * L)])
    pltpu.sync_copy(sw, w_out.at[pl.ds(base_elem, RPW * L)])
    pltpu.sync_copy(sh, h_out.at[pl.ds(base_elem, RPW * L)])


@functools.cache
def _make_sc_call():
    mesh = plsc.VectorSubcoreMesh(core_axis_name="c", subcore_axis_name="s",
                                  num_cores=NC, num_subcores=NS)
    f32 = jnp.float32
    out_type = (
        jax.ShapeDtypeStruct((ROWS * L,), f32),
        jax.ShapeDtypeStruct((ROWS * L,), jnp.int32),
        jax.ShapeDtypeStruct((ROWS * L,), f32),
        jax.ShapeDtypeStruct((ROWS * L,), f32),
        jax.ShapeDtypeStruct((ROWS * L,), f32),
        jax.ShapeDtypeStruct((ROWS * L,), f32),
    )
    scratch = [
        pltpu.VMEM((PADN,), f32),
        pltpu.VMEM((PADN,), f32),
        pltpu.VMEM((PADN,), f32),
        pltpu.VMEM((PADN,), f32),
        pltpu.VMEM((RPW * L,), f32),
        pltpu.VMEM((RPW * L,), f32),
        pltpu.VMEM((RPW * L,), f32),
        pltpu.VMEM((RPW * L,), jnp.int32),
        pltpu.VMEM((RPW * L,), f32),
        pltpu.VMEM((RPW * L,), f32),
        pltpu.VMEM((RPW * L,), f32),
        pltpu.VMEM((RPW * L,), f32),
        pltpu.VMEM((L,), f32),
        pltpu.VMEM((L,), jnp.int32),
        pltpu.VMEM((L,), f32),
        pltpu.VMEM((RPW * BROW,), f32),
        pltpu.VMEM((2 * L,), jnp.int32),
        pltpu.VMEM((RPW * L,), f32),
        pltpu.SemaphoreType.DMA,
        pltpu.SemaphoreType.DMA,
    ]
    cp = pltpu.CompilerParams()
    if "needs_layout_passes" in pltpu.CompilerParams.__dataclass_fields__:
        cp = dataclasses.replace(cp, needs_layout_passes=False)
    return pl.kernel(_sc_topk_body, out_type=out_type, mesh=mesh,
                     scratch_types=scratch, compiler_params=cp)


def _tc_finish_body(d2_ref, idx_ref, cx_ref, cy_ref, w_ref, h_ref, gt_ref,
                    kidx_ref, mask_ref, iou_ref):
    d2 = d2_ref[...]
    idx = idx_ref[...]
    cxg = cx_ref[...]
    cyg = cy_ref[...]
    wg = w_ref[...]
    hg = h_ref[...]

    d = jnp.sqrt(d2)
    inf = jnp.float32(jnp.inf)
    bigi = jnp.int32(2 ** 30)

    avail = jnp.ones(d.shape, jnp.bool_)
    sels = []
    for _ in range(K):
        dm = jnp.where(avail, d, inf)
        mn = jnp.min(dm, axis=1, keepdims=True)
        ismin = (dm == mn) & avail
        candi = jnp.where(ismin, idx, bigi)
        si = jnp.min(candi, axis=1, keepdims=True)
        sel = ismin & (idx == si)
        avail = avail & jnp.logical_not(sel)
        sels.append(sel)

    def pick_f(x, sel):
        return jnp.sum(jnp.where(sel, x, jnp.float32(0.0)), axis=1)[:, None]

    def pick_i(x, sel):
        return jnp.sum(jnp.where(sel, x, jnp.int32(0)), axis=1)[:, None]

    kidx = jnp.concatenate([pick_i(idx, s) for s in sels], axis=1)
    kcx = jnp.concatenate([pick_f(cxg, s) for s in sels], axis=1)
    kcy = jnp.concatenate([pick_f(cyg, s) for s in sels], axis=1)
    kw = jnp.concatenate([pick_f(wg, s) for s in sels], axis=1)
    kh = jnp.concatenate([pick_f(hg, s) for s in sels], axis=1)

    gcx = gt_ref[:, 0:1]
    gcy = gt_ref[:, 1:2]
    gw = gt_ref[:, 2:3]
    gh = gt_ref[:, 3:4]
    gx1 = gcx - 0.5 * gw
    gy1 = gcy - 0.5 * gh
    gx2 = gcx + 0.5 * gw
    gy2 = gcy + 0.5 * gh

    kx1 = kcx - 0.5 * kw
    ky1 = kcy - 0.5 * kh
    kx2 = kcx + 0.5 * kw
    ky2 = kcy + 0.5 * kh

    ltx = jnp.maximum(gx1, kx1)
    lty = jnp.maximum(gy1, ky1)
    rbx = jnp.minimum(gx2, kx2)
    rby = jnp.minimum(gy2, ky2)
    iw = jnp.clip(rbx - ltx, 0.0)
    ih = jnp.clip(rby - lty, 0.0)
    inter = iw * ih
    area_g = (gx2 - gx1) * (gy2 - gy1)
    area_k = (kx2 - kx1) * (ky2 - ky1)
    ious = inter / (area_g + area_k - inter + 1e-9)

    mean = jnp.mean(ious, axis=1, keepdims=True)
    var = jnp.sum((ious - mean) ** 2, axis=1, keepdims=True) / (K - 1)
    thr = mean + jnp.sqrt(var)

    mask = ((ious >= thr)
            & (gx1 <= kcx) & (kcx <= gx2)
            & (gy1 <= kcy) & (kcy <= gy2))

    kidx_ref[...] = kidx
    mask_ref[...] = mask.astype(jnp.int32)
    iou_ref[...] = ious


@functools.cache
def _make_tc_call():
    out_shape = (
        jax.ShapeDtypeStruct((ROWS, K), jnp.int32),
        jax.ShapeDtypeStruct((ROWS, K), jnp.int32),
        jax.ShapeDtypeStruct((ROWS, K), jnp.float32),
    )
    return pl.pallas_call(_tc_finish_body, out_shape=out_shape)


@jax.jit
def kernel(pred_boxes, gt_boxes):
    prep = jnp.pad(jnp.moveaxis(pred_boxes, 2, 1),
                   ((0, 0), (0, 0), (0, PADN - N)), constant_values=1e9)
    gx = jnp.broadcast_to(gt_boxes[..., 0].reshape(ROWS, 1), (ROWS, L)).reshape(-1)
    gy = jnp.broadcast_to(gt_boxes[..., 1].reshape(ROWS, 1), (ROWS, L)).reshape(-1)

    d2, idx, kcx, kcy, kw, kh = _make_sc_call()(prep, gx, gy)

    gtf = gt_boxes.reshape(ROWS, 4)
    kidx, mask_i, ious = _make_tc_call()(
        d2.reshape(ROWS, L), idx.reshape(ROWS, L),
        kcx.reshape(ROWS, L), kcy.reshape(ROWS, L),
        kw.reshape(ROWS, L), kh.reshape(ROWS, L), gtf)

    return (kidx.reshape(B, G, K),
            mask_i.astype(jnp.bool_).reshape(B, G, K),
            ious.reshape(B, G, K))


# speedup vs baseline: 1.1537x; 1.1537x over previous
import dataclasses
import functools

import jax
import jax.numpy as jnp
from jax import lax
from jax.experimental import pallas as pl
from jax.experimental.pallas import tpu as pltpu
from jax.experimental.pallas import tpu_sc as plsc

K = 9
L = 16
B = 4
N = 20000
G = 64
ROWS = B * G
NC = 2
NS = 16
NW = NC * NS
RPW = ROWS // NW
WPB = NW // B
CHUNKS = N // L
BCH = 16
NBLK = (CHUNKS + BCH - 1) // BCH
PADN = NBLK * BCH * L
NSTR = NBLK * L
GRB = 8
BROW = (NBLK + 1) * L


def _dyn_gather(x, idx):
    dnums = lax.GatherDimensionNumbers(
        offset_dims=(), collapsed_slice_dims=(0,), start_index_map=(0,))
    return lax.gather(x, idx[:, None], dnums, (1,),
                      mode=lax.GatherScatterMode.PROMISE_IN_BOUNDS)


def _sc_topk_body(prep_hbm, gx_hbm, gy_hbm,
                  d2_out, idx_out, cx_out, cy_out, w_out, h_out,
                  cx_v, cy_v, w_v, h_v, gx_v, gy_v,
                  sd2, sidx, scx, scy, sw, sh,
                  top_d_ref, top_i_ref, tmax_ref, bmin_ref, sid_ref,
                  gmin_ref, sem_w, sem_h):
    cid = lax.axis_index("c")
    sid = lax.axis_index("s")
    wid = cid * NS + sid
    b = wid // WPB
    base_elem = wid * (RPW * L)

    cp_w = pltpu.async_copy(prep_hbm.at[b, 2], w_v, sem_w)
    cp_h = pltpu.async_copy(prep_hbm.at[b, 3], h_v, sem_h)
    pltpu.sync_copy(prep_hbm.at[b, 0], cx_v)
    pltpu.sync_copy(prep_hbm.at[b, 1], cy_v)
    pltpu.sync_copy(gx_hbm.at[pl.ds(base_elem, RPW * L)], gx_v)
    pltpu.sync_copy(gy_hbm.at[pl.ds(base_elem, RPW * L)], gy_v)

    lanes = lax.iota(jnp.int32, L)
    last_lane = jnp.full((L,), L - 1, jnp.int32)

    def sort_merge(d2, idxc):
        td = top_d_ref[...]
        ti = top_i_ref[...]
        ds_, is_ = plsc.sort_key_val(d2, idxc)
        rd = lax.rev(ds_, (0,))
        ri = lax.rev(is_, (0,))
        takes = td <= rd
        md = jnp.where(takes, td, rd)
        mi = jnp.where(takes, ti, ri)
        nd, ni = plsc.sort_key_val(md, mi)
        top_d_ref[...] = nd
        top_i_ref[...] = ni
        tmax_ref[...] = _dyn_gather(nd, last_lane)

    gxs = [gx_v[pl.ds(g * L, L)] for g in range(RPW)]
    gys = [gy_v[pl.ds(g * L, L)] for g in range(RPW)]
    inf16 = jnp.full((L,), jnp.inf, jnp.float32)

    @pl.loop(0, NBLK, init_carry=tuple(inf16 for _ in range(RPW)))
    def _per_block(k, gmins):
        base0 = k * (BCH * L)
        accs = [None] * RPW
        for c in range(BCH):
            cxc = cx_v[pl.ds(base0 + c * L, L)]
            cyc = cy_v[pl.ds(base0 + c * L, L)]
            for g in range(RPW):
                dx = cxc - gxs[g]
                dy = cyc - gys[g]
                d2 = dx * dx + dy * dy
                accs[g] = d2 if c == 0 else jnp.minimum(accs[g], d2)
        for g in range(RPW):
            bmin_ref[pl.ds(g * BROW + k * L, L)] = accs[g]
        return tuple(jnp.minimum(gm, acc) for gm, acc in zip(gmins, accs))

    gmins = _per_block
    for g in range(RPW):
        gmin_ref[pl.ds(g * L, L)] = gmins[g]
        bmin_ref[pl.ds(g * BROW + NBLK * L, L)] = inf16
    cp_w.wait()
    cp_h.wait()

    @pl.loop(0, RPW)
    def _per_gt(g):
        gxv = gx_v[pl.ds(g * L, L)]
        gyv = gy_v[pl.ds(g * L, L)]
        bbase = g * BROW

        gmin = gmin_ref[pl.ds(g * L, L)]
        seed = plsc.bitcast(
            plsc.bitcast(jnp.full((L,), jnp.max(gmin), jnp.float32),
                         jnp.int32) + 1, jnp.float32)
        top_d_ref[...] = jnp.full((L,), jnp.inf, jnp.float32)
        top_i_ref[...] = jnp.zeros((L,), jnp.int32)
        tmax_ref[...] = seed

        @pl.loop(0, (NBLK + 1) // GRB)
        def _scan_bmin(gi):
            b0 = gi * (GRB * L)
            tm = tmax_ref[...]
            hit = jnp.zeros((L,), jnp.bool_)
            for jj in range(GRB):
                bm = bmin_ref[pl.ds(bbase + b0 + jj * L, L)]
                hit = hit | (bm < tm)

            @pl.when(jnp.any(hit))
            def _():
                for jj in range(GRB):
                    bm = bmin_ref[pl.ds(bbase + b0 + jj * L, L)]

                    @pl.when(jnp.any(bm < tmax_ref[...]))
                    def _():
                        sort_merge(bm, b0 + jj * L + lanes)

        sid_ref[pl.ds(0, L)] = top_i_ref[...]
        sid_ref[pl.ds(L, L)] = top_i_ref[...]
        top_d_ref[...] = jnp.full((L,), jnp.inf, jnp.float32)
        top_i_ref[...] = jnp.zeros((L,), jnp.int32)
        tmax_ref[...] = jnp.full((L,), jnp.inf, jnp.float32)

        for j in range(L):
            sv = plsc.load_gather(sid_ref, [jnp.full((L,), L + j, jnp.int32)])
            blk = lax.shift_right_logical(sv, 4)
            lane = jnp.bitwise_and(sv, L - 1)
            eidx = blk * (BCH * L) + lane + lanes * L
            cxg = plsc.load_gather(cx_v, [eidx])
            cyg = plsc.load_gather(cy_v, [eidx])
            dx = cxg - gxv
            dy = cyg - gyv
            d2 = dx * dx + dy * dy

            @pl.when(jnp.any(d2 < tmax_ref[...]))
            def _():
                sort_merge(d2, eidx)

        top_d = top_d_ref[...]
        top_i = top_i_ref[...]
        o = g * L
        sd2[pl.ds(o, L)] = top_d
        sidx[pl.ds(o, L)] = top_i
        scx[pl.ds(o, L)] = plsc.load_gather(cx_v, [top_i])
        scy[pl.ds(o, L)] = plsc.load_gather(cy_v, [top_i])
        sw[pl.ds(o, L)] = plsc.load_gather(w_v, [top_i])
        sh[pl.ds(o, L)] = plsc.load_gather(h_v, [top_i])

    pltpu.sync_copy(sd2, d2_out.at[pl.ds(base_elem, RPW * L)])
    pltpu.sync_copy(sidx, idx_out.at[pl.ds(base_elem, RPW * L)])
    pltpu.sync_copy(scx, cx_out.at[pl.ds(base_elem, RPW * L)])
    pltpu.sync_copy(scy, cy_out.at[pl.ds(base_elem, RPW * L)])
    pltpu.sync_copy(sw, w_out.at[pl.ds(base_elem, RPW * L)])
    pltpu.sync_copy(sh, h_out.at[pl.ds(base_elem, RPW * L)])


@functools.cache
def _make_sc_call():
    mesh = plsc.VectorSubcoreMesh(core_axis_name="c", subcore_axis_name="s",
                                  num_cores=NC, num_subcores=NS)
    f32 = jnp.float32
    out_type = (
        jax.ShapeDtypeStruct((ROWS * L,), f32),
        jax.ShapeDtypeStruct((ROWS * L,), jnp.int32),
        jax.ShapeDtypeStruct((ROWS * L,), f32),
        jax.ShapeDtypeStruct((ROWS * L,), f32),
        jax.ShapeDtypeStruct((ROWS * L,), f32),
        jax.ShapeDtypeStruct((ROWS * L,), f32),
    )
    scratch = [
        pltpu.VMEM((PADN,), f32),
        pltpu.VMEM((PADN,), f32),
        pltpu.VMEM((PADN,), f32),
        pltpu.VMEM((PADN,), f32),
        pltpu.VMEM((RPW * L,), f32),
        pltpu.VMEM((RPW * L,), f32),
        pltpu.VMEM((RPW * L,), f32),
        pltpu.VMEM((RPW * L,), jnp.int32),
        pltpu.VMEM((RPW * L,), f32),
        pltpu.VMEM((RPW * L,), f32),
        pltpu.VMEM((RPW * L,), f32),
        pltpu.VMEM((RPW * L,), f32),
        pltpu.VMEM((L,), f32),
        pltpu.VMEM((L,), jnp.int32),
        pltpu.VMEM((L,), f32),
        pltpu.VMEM((RPW * BROW,), f32),
        pltpu.VMEM((2 * L,), jnp.int32),
        pltpu.VMEM((RPW * L,), f32),
        pltpu.SemaphoreType.DMA,
        pltpu.SemaphoreType.DMA,
    ]
    cp = pltpu.CompilerParams()
    if "needs_layout_passes" in pltpu.CompilerParams.__dataclass_fields__:
        cp = dataclasses.replace(cp, needs_layout_passes=False)
    return pl.kernel(_sc_topk_body, out_type=out_type, mesh=mesh,
                     scratch_types=scratch, compiler_params=cp)


def _tc_finish_body(d2_ref, idx_ref, cx_ref, cy_ref, w_ref, h_ref, gt_ref,
                    kidx_ref, mask_ref, iou_ref):
    d2 = d2_ref[...]
    idx = idx_ref[...]
    cxg = cx_ref[...]
    cyg = cy_ref[...]
    wg = w_ref[...]
    hg = h_ref[...]

    d = jnp.sqrt(d2)
    inf = jnp.float32(jnp.inf)
    bigi = jnp.int32(2 ** 30)

    avail = jnp.ones(d.shape, jnp.bool_)
    sels = []
    for _ in range(K):
        dm = jnp.where(avail, d, inf)
        mn = jnp.min(dm, axis=1, keepdims=True)
        ismin = (dm == mn) & avail
        candi = jnp.where(ismin, idx, bigi)
        si = jnp.min(candi, axis=1, keepdims=True)
        sel = ismin & (idx == si)
        avail = avail & jnp.logical_not(sel)
        sels.append(sel)

    def pick_f(x, sel):
        return jnp.sum(jnp.where(sel, x, jnp.float32(0.0)), axis=1)[:, None]

    def pick_i(x, sel):
        return jnp.sum(jnp.where(sel, x, jnp.int32(0)), axis=1)[:, None]

    kidx = jnp.concatenate([pick_i(idx, s) for s in sels], axis=1)
    kcx = jnp.concatenate([pick_f(cxg, s) for s in sels], axis=1)
    kcy = jnp.concatenate([pick_f(cyg, s) for s in sels], axis=1)
    kw = jnp.concatenate([pick_f(wg, s) for s in sels], axis=1)
    kh = jnp.concatenate([pick_f(hg, s) for s in sels], axis=1)

    gcx = gt_ref[:, 0:1]
    gcy = gt_ref[:, 1:2]
    gw = gt_ref[:, 2:3]
    gh = gt_ref[:, 3:4]
    gx1 = gcx - 0.5 * gw
    gy1 = gcy - 0.5 * gh
    gx2 = gcx + 0.5 * gw
    gy2 = gcy + 0.5 * gh

    kx1 = kcx - 0.5 * kw
    ky1 = kcy - 0.5 * kh
    kx2 = kcx + 0.5 * kw
    ky2 = kcy + 0.5 * kh

    ltx = jnp.maximum(gx1, kx1)
    lty = jnp.maximum(gy1, ky1)
    rbx = jnp.minimum(gx2, kx2)
    rby = jnp.minimum(gy2, ky2)
    iw = jnp.clip(rbx - ltx, 0.0)
    ih = jnp.clip(rby - lty, 0.0)
    inter = iw * ih
    area_g = (gx2 - gx1) * (gy2 - gy1)
    area_k = (kx2 - kx1) * (ky2 - ky1)
    ious = inter / (area_g + area_k - inter + 1e-9)

    mean = jnp.mean(ious, axis=1, keepdims=True)
    var = jnp.sum((ious - mean) ** 2, axis=1, keepdims=True) / (K - 1)
    thr = mean + jnp.sqrt(var)

    mask = ((ious >= thr)
            & (gx1 <= kcx) & (kcx <= gx2)
            & (gy1 <= kcy) & (kcy <= gy2))

    kidx_ref[...] = kidx
    mask_ref[...] = mask.astype(jnp.int32)
    iou_ref[...] = ious


@functools.cache
def _make_tc_call():
    out_shape = (
        jax.ShapeDtypeStruct((ROWS, K), jnp.int32),
        jax.ShapeDtypeStruct((ROWS, K), jnp.int32),
        jax.ShapeDtypeStruct((ROWS, K), jnp.float32),
    )
    return pl.pallas_call(_tc_finish_body, out_shape=out_shape)


@jax.jit
def kernel(pred_boxes, gt_boxes):
    prep = jnp.pad(jnp.moveaxis(pred_boxes, 2, 1),
                   ((0, 0), (0, 0), (0, PADN - N)), constant_values=1e9)
    gx = jnp.broadcast_to(gt_boxes[..., 0].reshape(ROWS, 1), (ROWS, L)).reshape(-1)
    gy = jnp.broadcast_to(gt_boxes[..., 1].reshape(ROWS, 1), (ROWS, L)).reshape(-1)

    d2, idx, kcx, kcy, kw, kh = _make_sc_call()(prep, gx, gy)

    gtf = gt_boxes.reshape(ROWS, 4)
    kidx, mask_i, ious = _make_tc_call()(
        d2.reshape(ROWS, L), idx.reshape(ROWS, L),
        kcx.reshape(ROWS, L), kcy.reshape(ROWS, L),
        kw.reshape(ROWS, L), kh.reshape(ROWS, L), gtf)

    return (kidx.reshape(B, G, K),
            mask_i.astype(jnp.bool_).reshape(B, G, K),
            ious.reshape(B, G, K))
